# baseline (speedup 1.0000x reference)
import functools
import jax
import jax.numpy as jnp
from jax import lax
from jax.experimental import pallas as pl
from jax.experimental.pallas import tpu as pltpu
from jax.experimental.pallas import tpu_sc as plsc

N = 10000
E = 160000
D = 256
H = 256
C = 40

NPAD = 10240
B = 128
G = 16
SUB = 16
ROWS_PER_SUB = NPAD // SUB
EP = 2 * SUB * B * ((E + 2 * SUB * B - 1) // (2 * SUB * B))
CH1 = EP // (SUB * B)
CH2 = EP // (2 * SUB * B)
W1CH = 16

_mesh = plsc.VectorSubcoreMesh(core_axis_name="c", subcore_axis_name="s")


def _zero_slice(acc, buf, base):
    for k in range(ROWS_PER_SUB // B):
        pltpu.sync_copy(buf, acc.at[pl.ds(base + k * B, B)])


def _scatter_chunk_async(buf, acc, dst_w, jj, ssem):
    out = []
    for g in range(B // G):
        iv = dst_w[jj, pl.ds(g * G, G)]
        out.append(pltpu.async_copy(buf.at[pl.ds(g * G, G)], acc.at[iv],
                                    ssem, add=True))
    return out


def _agg_window(table, src_rows, dst_rows, acc, src_w, dst_w, buf,
                gsem, ssem, w, wch):
    pltpu.sync_copy(src_rows.at[pl.ds(w * wch, wch)], src_w)
    pltpu.sync_copy(dst_rows.at[pl.ds(w * wch, wch)], dst_w)
    gd = pltpu.async_copy(table.at[src_w.at[0]], buf.at[0], gsem)
    pend = []
    for jj in range(wch):
        gd.wait()
        if jj + 1 < wch:
            for d in pend:
                d.wait()
            gd = pltpu.async_copy(table.at[src_w.at[jj + 1]],
                                  buf.at[(jj + 1) % 2], gsem)
        sc = _scatter_chunk_async(buf.at[jj % 2], acc, dst_w, jj, ssem)
        if jj + 1 < wch:
            pend = sc
        else:
            for d in pend:
                d.wait()
            for d in sc:
                d.wait()


@functools.partial(
    pl.kernel,
    out_type=[jax.ShapeDtypeStruct((NPAD, 128), jnp.float32),
              jax.ShapeDtypeStruct((NPAD, 128), jnp.float32)],
    mesh=_mesh,
    scratch_types=[
        pltpu.VMEM((CH2, B), jnp.int32),
        pltpu.VMEM((B, 128), jnp.float32),
        pltpu.VMEM_SHARED((NPAD, 128), jnp.float32),
        pltpu.SemaphoreType.DMA,
    ],
)
def _deg_kernel(dst_hbm, ones_hbm, zeros_hbm, deg0_hbm, deg1_hbm,
                dst_v, buf, acc, ssem):
    c = lax.axis_index("c")
    s = lax.axis_index("s")
    w = c * SUB + s
    pltpu.sync_copy(zeros_hbm, buf)
    pltpu.sync_copy(dst_hbm.at[w], dst_v)
    base = s * ROWS_PER_SUB
    _zero_slice(acc, buf, base)
    plsc.subcore_barrier()
    pltpu.sync_copy(ones_hbm, buf)
    pend = []
    for jj in range(CH2):
        sc = _scatter_chunk_async(buf, acc, dst_v, jj, ssem)
        for d in pend:
            d.wait()
        pend = sc
    for d in pend:
        d.wait()
    plsc.subcore_barrier()

    @pl.when(c == 0)
    def _():
        pltpu.sync_copy(acc.at[pl.ds(base, ROWS_PER_SUB)],
                        deg0_hbm.at[pl.ds(base, ROWS_PER_SUB)])

    @pl.when(c == 1)
    def _():
        pltpu.sync_copy(acc.at[pl.ds(base, ROWS_PER_SUB)],
                        deg1_hbm.at[pl.ds(base, ROWS_PER_SUB)])


@functools.partial(
    pl.kernel,
    out_type=[jax.ShapeDtypeStruct((NPAD, 128), jnp.float32),
              jax.ShapeDtypeStruct((NPAD, 128), jnp.float32)],
    mesh=_mesh,
    scratch_types=[
        pltpu.VMEM((W1CH, B), jnp.int32),
        pltpu.VMEM((W1CH, B), jnp.int32),
        pltpu.VMEM((2, B, 128), jnp.float32),
        pltpu.VMEM_SHARED((NPAD, 128), jnp.float32),
        pltpu.SemaphoreType.DMA,
        pltpu.SemaphoreType.DMA,
    ],
)
def _agg1_kernel(h0_hbm, h1_hbm, src_hbm, dst_hbm,
                 a0_hbm, a1_hbm, src_w, dst_w, buf, acc, gsem, ssem):
    c = lax.axis_index("c")
    s = lax.axis_index("s")
    base = s * ROWS_PER_SUB

    @pl.when(c == 0)
    def _():
        pltpu.sync_copy(h0_hbm.at[pl.ds(base, ROWS_PER_SUB)],
                        acc.at[pl.ds(base, ROWS_PER_SUB)])

    @pl.when(c == 1)
    def _():
        pltpu.sync_copy(h1_hbm.at[pl.ds(base, ROWS_PER_SUB)],
                        acc.at[pl.ds(base, ROWS_PER_SUB)])
    plsc.subcore_barrier()

    @pl.when(c == 0)
    def _():
        def body(w, carry):
            _agg_window(h0_hbm, src_hbm.at[s], dst_hbm.at[s], acc,
                        src_w, dst_w, buf, gsem, ssem, w, W1CH)
            return carry
        lax.fori_loop(0, CH1 // W1CH, body, 0)

    @pl.when(c == 1)
    def _():
        def body(w, carry):
            _agg_window(h1_hbm, src_hbm.at[s], dst_hbm.at[s], acc,
                        src_w, dst_w, buf, gsem, ssem, w, W1CH)
            return carry
        lax.fori_loop(0, CH1 // W1CH, body, 0)

    plsc.subcore_barrier()

    @pl.when(c == 0)
    def _():
        pltpu.sync_copy(acc.at[pl.ds(base, ROWS_PER_SUB)],
                        a0_hbm.at[pl.ds(base, ROWS_PER_SUB)])

    @pl.when(c == 1)
    def _():
        pltpu.sync_copy(acc.at[pl.ds(base, ROWS_PER_SUB)],
                        a1_hbm.at[pl.ds(base, ROWS_PER_SUB)])


@functools.partial(
    pl.kernel,
    out_type=[jax.ShapeDtypeStruct((NPAD, 128), jnp.float32),
              jax.ShapeDtypeStruct((NPAD, 128), jnp.float32)],
    mesh=_mesh,
    scratch_types=[
        pltpu.VMEM((CH2, B), jnp.int32),
        pltpu.VMEM((CH2, B), jnp.int32),
        pltpu.VMEM((2, B, 128), jnp.float32),
        pltpu.VMEM_SHARED((NPAD, 128), jnp.float32),
        pltpu.SemaphoreType.DMA,
        pltpu.SemaphoreType.DMA,
    ],
)
def _agg2_kernel(h_hbm, src_hbm, dst_hbm, zeros_hbm,
                 p0_hbm, p1_hbm, src_w, dst_w, buf, acc, gsem, ssem):
    c = lax.axis_index("c")
    s = lax.axis_index("s")
    w = c * SUB + s
    base = s * ROWS_PER_SUB

    @pl.when(c == 0)
    def _():
        pltpu.sync_copy(h_hbm.at[pl.ds(base, ROWS_PER_SUB)],
                        acc.at[pl.ds(base, ROWS_PER_SUB)])

    @pl.when(c == 1)
    def _():
        pltpu.sync_copy(zeros_hbm, buf.at[0])
        _zero_slice(acc, buf.at[0], base)
    plsc.subcore_barrier()
    _agg_window(h_hbm, src_hbm.at[w], dst_hbm.at[w], acc,
                src_w, dst_w, buf, gsem, ssem, 0, CH2)
    plsc.subcore_barrier()

    @pl.when(c == 0)
    def _():
        pltpu.sync_copy(acc.at[pl.ds(base, ROWS_PER_SUB)],
                        p0_hbm.at[pl.ds(base, ROWS_PER_SUB)])

    @pl.when(c == 1)
    def _():
        pltpu.sync_copy(acc.at[pl.ds(base, ROWS_PER_SUB)],
                        p1_hbm.at[pl.ds(base, ROWS_PER_SUB)])


_BR = 1024


def _mm0_body(x_ref, w_ref, m_ref):
    m_ref[...] = jnp.dot(x_ref[...], w_ref[...],
                         preferred_element_type=jnp.float32)


def _scale_body(m_ref, d0_ref, d1_ref, h0_ref, h1_ref):
    dinv = lax.rsqrt(d0_ref[...] + d1_ref[...] + 1.0)
    h = m_ref[...]
    h0_ref[...] = h[:, :128] * dinv
    h1_ref[...] = h[:, 128:] * dinv


def _mm2_body(a0_ref, a1_ref, d0_ref, d1_ref, w_ref, b1_ref, out_ref):
    dinv = lax.rsqrt(d0_ref[...] + d1_ref[...] + 1.0)
    z = jnp.concatenate(
        [a0_ref[...] * dinv, a1_ref[...] * dinv], axis=1)
    z = jnp.maximum(z + b1_ref[...], 0.0)
    out_ref[...] = jnp.dot(
        z, w_ref[...], preferred_element_type=jnp.float32) * dinv


def _out_body(p0_ref, p1_ref, d0_ref, d1_ref, b2_ref, out_ref):
    dinv = lax.rsqrt(d0_ref[...] + d1_ref[...] + 1.0)
    o = (p0_ref[...] + p1_ref[...]) * dinv + b2_ref[...]
    col = lax.broadcasted_iota(jnp.int32, o.shape, 1)
    valid = col < C
    om = jnp.where(valid, o, -3e38)
    m = jnp.max(om, axis=1, keepdims=True)
    ex = jnp.where(valid, jnp.exp(o - m), 0.0)
    lse = jnp.log(jnp.sum(ex, axis=1, keepdims=True))
    out_ref[...] = o - m - lse


def _row_spec(width):
    return pl.BlockSpec((_BR, width), lambda i: (i, 0))


def _full_spec(shape):
    return pl.BlockSpec(shape, lambda i: tuple(0 for _ in shape))


def kernel(x, edge_index, W1, b1, W2, b2):
    f32 = jnp.float32
    x_pad = jnp.zeros((NPAD, D), f32).at[:N].set(x)
    W2p = jnp.zeros((H, 128), f32).at[:, :C].set(W2)
    b1r = b1.reshape(1, H)
    b2p = jnp.zeros((1, 128), f32).at[0, :C].set(b2)

    pad = N + jnp.arange(EP - E, dtype=jnp.int32) % (NPAD - N)
    src_flat = jnp.concatenate([edge_index[0], pad])
    dst_flat = jnp.concatenate([edge_index[1], pad])
    src1 = src_flat.reshape(SUB, CH1, B)
    dst1 = dst_flat.reshape(SUB, CH1, B)
    src2 = src_flat.reshape(2 * SUB, CH2, B)
    dst2 = dst_flat.reshape(2 * SUB, CH2, B)

    ones128 = jnp.ones((B, 128), f32)
    zer128 = jnp.zeros((B, 128), f32)

    m = pl.pallas_call(
        _mm0_body,
        grid=(NPAD // _BR,),
        in_specs=[_row_spec(D), _full_spec((D, H))],
        out_specs=_row_spec(H),
        out_shape=jax.ShapeDtypeStruct((NPAD, H), f32),
    )(x_pad, W1)

    deg0, deg1 = _deg_kernel(dst2, ones128, zer128)

    h0, h1 = pl.pallas_call(
        _scale_body,
        grid=(NPAD // _BR,),
        in_specs=[_row_spec(H), _row_spec(128), _row_spec(128)],
        out_specs=[_row_spec(128), _row_spec(128)],
        out_shape=[jax.ShapeDtypeStruct((NPAD, 128), f32),
                   jax.ShapeDtypeStruct((NPAD, 128), f32)],
    )(m, deg0, deg1)

    a0, a1 = _agg1_kernel(h0, h1, src1, dst1)

    h2 = pl.pallas_call(
        _mm2_body,
        grid=(NPAD // _BR,),
        in_specs=[_row_spec(128), _row_spec(128), _row_spec(128),
                  _row_spec(128), _full_spec((H, 128)), _full_spec((1, H))],
        out_specs=_row_spec(128),
        out_shape=jax.ShapeDtypeStruct((NPAD, 128), f32),
    )(a0, a1, deg0, deg1, W2p, b1r)

    p0, p1 = _agg2_kernel(h2, src2, dst2, zer128)

    lsm = pl.pallas_call(
        _out_body,
        grid=(NPAD // _BR,),
        in_specs=[_row_spec(128), _row_spec(128), _row_spec(128),
                  _row_spec(128), _full_spec((1, 128))],
        out_specs=_row_spec(128),
        out_shape=jax.ShapeDtypeStruct((NPAD, 128), f32),
    )(p0, p1, deg0, deg1, b2p)

    return lsm[:N, :C]


# speedup vs baseline: 1.0105x; 1.0105x over previous
import functools
import jax
import jax.numpy as jnp
from jax import lax
from jax.experimental import pallas as pl
from jax.experimental.pallas import tpu as pltpu
from jax.experimental.pallas import tpu_sc as plsc

N = 10000
E = 160000
D = 256
H = 256
C = 40

NPAD = 10240
B = 128
G = 16
SUB = 16
ROWS_PER_SUB = NPAD // SUB
EP = 2 * SUB * B * ((E + 2 * SUB * B - 1) // (2 * SUB * B))
CH1 = EP // (SUB * B)
CH2 = EP // (2 * SUB * B)
W1CH = 40

_mesh = plsc.VectorSubcoreMesh(core_axis_name="c", subcore_axis_name="s")


def _zero_slice(acc, buf, base):
    for k in range(ROWS_PER_SUB // B):
        pltpu.sync_copy(buf, acc.at[pl.ds(base + k * B, B)])


def _scatter_chunk_async(buf, acc, dst_w, jj, ssem):
    out = []
    for g in range(B // G):
        iv = dst_w[jj, pl.ds(g * G, G)]
        out.append(pltpu.async_copy(buf.at[pl.ds(g * G, G)], acc.at[iv],
                                    ssem, add=True))
    return out


def _agg_window(table, src_rows, dst_rows, acc, src_w, dst_w, buf,
                gsem, ssem, w, wch):
    pltpu.sync_copy(src_rows.at[pl.ds(w * wch, wch)], src_w)
    pltpu.sync_copy(dst_rows.at[pl.ds(w * wch, wch)], dst_w)
    gd = pltpu.async_copy(table.at[src_w.at[0]], buf.at[0], gsem)
    pend = []
    for jj in range(wch):
        gd.wait()
        if jj + 1 < wch:
            for d in pend:
                d.wait()
            gd = pltpu.async_copy(table.at[src_w.at[jj + 1]],
                                  buf.at[(jj + 1) % 2], gsem)
        sc = _scatter_chunk_async(buf.at[jj % 2], acc, dst_w, jj, ssem)
        if jj + 1 < wch:
            pend = sc
        else:
            for d in pend:
                d.wait()
            for d in sc:
                d.wait()


@functools.partial(
    pl.kernel,
    out_type=[jax.ShapeDtypeStruct((NPAD, 128), jnp.float32),
              jax.ShapeDtypeStruct((NPAD, 128), jnp.float32)],
    mesh=_mesh,
    scratch_types=[
        pltpu.VMEM((CH2, B), jnp.int32),
        pltpu.VMEM((B, 128), jnp.float32),
        pltpu.VMEM_SHARED((NPAD, 128), jnp.float32),
        pltpu.SemaphoreType.DMA,
    ],
)
def _deg_kernel(dst_hbm, ones_hbm, zeros_hbm, deg0_hbm, deg1_hbm,
                dst_v, buf, acc, ssem):
    c = lax.axis_index("c")
    s = lax.axis_index("s")
    w = c * SUB + s
    pltpu.sync_copy(zeros_hbm, buf)
    pltpu.sync_copy(dst_hbm.at[w], dst_v)
    base = s * ROWS_PER_SUB
    _zero_slice(acc, buf, base)
    plsc.subcore_barrier()
    pltpu.sync_copy(ones_hbm, buf)
    pend = []
    for jj in range(CH2):
        sc = _scatter_chunk_async(buf, acc, dst_v, jj, ssem)
        for d in pend:
            d.wait()
        pend = sc
    for d in pend:
        d.wait()
    plsc.subcore_barrier()

    @pl.when(c == 0)
    def _():
        pltpu.sync_copy(acc.at[pl.ds(base, ROWS_PER_SUB)],
                        deg0_hbm.at[pl.ds(base, ROWS_PER_SUB)])

    @pl.when(c == 1)
    def _():
        pltpu.sync_copy(acc.at[pl.ds(base, ROWS_PER_SUB)],
                        deg1_hbm.at[pl.ds(base, ROWS_PER_SUB)])


@functools.partial(
    pl.kernel,
    out_type=[jax.ShapeDtypeStruct((NPAD, 128), jnp.float32),
              jax.ShapeDtypeStruct((NPAD, 128), jnp.float32)],
    mesh=_mesh,
    scratch_types=[
        pltpu.VMEM((W1CH, B), jnp.int32),
        pltpu.VMEM((W1CH, B), jnp.int32),
        pltpu.VMEM((2, B, 128), jnp.float32),
        pltpu.VMEM_SHARED((NPAD, 128), jnp.float32),
        pltpu.SemaphoreType.DMA,
        pltpu.SemaphoreType.DMA,
    ],
)
def _agg1_kernel(h0_hbm, h1_hbm, src_hbm, dst_hbm,
                 a0_hbm, a1_hbm, src_w, dst_w, buf, acc, gsem, ssem):
    c = lax.axis_index("c")
    s = lax.axis_index("s")
    base = s * ROWS_PER_SUB

    @pl.when(c == 0)
    def _():
        pltpu.sync_copy(h0_hbm.at[pl.ds(base, ROWS_PER_SUB)],
                        acc.at[pl.ds(base, ROWS_PER_SUB)])

    @pl.when(c == 1)
    def _():
        pltpu.sync_copy(h1_hbm.at[pl.ds(base, ROWS_PER_SUB)],
                        acc.at[pl.ds(base, ROWS_PER_SUB)])
    plsc.subcore_barrier()

    @pl.when(c == 0)
    def _():
        def body(w, carry):
            _agg_window(h0_hbm, src_hbm.at[s], dst_hbm.at[s], acc,
                        src_w, dst_w, buf, gsem, ssem, w, W1CH)
            return carry
        lax.fori_loop(0, CH1 // W1CH, body, 0)

    @pl.when(c == 1)
    def _():
        def body(w, carry):
            _agg_window(h1_hbm, src_hbm.at[s], dst_hbm.at[s], acc,
                        src_w, dst_w, buf, gsem, ssem, w, W1CH)
            return carry
        lax.fori_loop(0, CH1 // W1CH, body, 0)

    plsc.subcore_barrier()

    @pl.when(c == 0)
    def _():
        pltpu.sync_copy(acc.at[pl.ds(base, ROWS_PER_SUB)],
                        a0_hbm.at[pl.ds(base, ROWS_PER_SUB)])

    @pl.when(c == 1)
    def _():
        pltpu.sync_copy(acc.at[pl.ds(base, ROWS_PER_SUB)],
                        a1_hbm.at[pl.ds(base, ROWS_PER_SUB)])


@functools.partial(
    pl.kernel,
    out_type=[jax.ShapeDtypeStruct((NPAD, 128), jnp.float32),
              jax.ShapeDtypeStruct((NPAD, 128), jnp.float32)],
    mesh=_mesh,
    scratch_types=[
        pltpu.VMEM((CH2, B), jnp.int32),
        pltpu.VMEM((CH2, B), jnp.int32),
        pltpu.VMEM((2, B, 128), jnp.float32),
        pltpu.VMEM_SHARED((NPAD, 128), jnp.float32),
        pltpu.SemaphoreType.DMA,
        pltpu.SemaphoreType.DMA,
    ],
)
def _agg2_kernel(h_hbm, src_hbm, dst_hbm, zeros_hbm,
                 p0_hbm, p1_hbm, src_w, dst_w, buf, acc, gsem, ssem):
    c = lax.axis_index("c")
    s = lax.axis_index("s")
    w = c * SUB + s
    base = s * ROWS_PER_SUB

    @pl.when(c == 0)
    def _():
        pltpu.sync_copy(h_hbm.at[pl.ds(base, ROWS_PER_SUB)],
                        acc.at[pl.ds(base, ROWS_PER_SUB)])

    @pl.when(c == 1)
    def _():
        pltpu.sync_copy(zeros_hbm, buf.at[0])
        _zero_slice(acc, buf.at[0], base)
    plsc.subcore_barrier()
    _agg_window(h_hbm, src_hbm.at[w], dst_hbm.at[w], acc,
                src_w, dst_w, buf, gsem, ssem, 0, CH2)
    plsc.subcore_barrier()

    @pl.when(c == 0)
    def _():
        pltpu.sync_copy(acc.at[pl.ds(base, ROWS_PER_SUB)],
                        p0_hbm.at[pl.ds(base, ROWS_PER_SUB)])

    @pl.when(c == 1)
    def _():
        pltpu.sync_copy(acc.at[pl.ds(base, ROWS_PER_SUB)],
                        p1_hbm.at[pl.ds(base, ROWS_PER_SUB)])


_BR = 1024


def _mm1_body(x_ref, w_ref, d0_ref, d1_ref, h0_ref, h1_ref):
    dinv = lax.rsqrt(d0_ref[...] + d1_ref[...] + 1.0)
    h = jnp.dot(x_ref[...], w_ref[...], preferred_element_type=jnp.float32)
    h0_ref[...] = h[:, :128] * dinv
    h1_ref[...] = h[:, 128:] * dinv


def _mm2_body(a0_ref, a1_ref, d0_ref, d1_ref, w_ref, b1_ref, out_ref):
    dinv = lax.rsqrt(d0_ref[...] + d1_ref[...] + 1.0)
    z = jnp.concatenate(
        [a0_ref[...] * dinv, a1_ref[...] * dinv], axis=1)
    z = jnp.maximum(z + b1_ref[...], 0.0)
    out_ref[...] = jnp.dot(
        z, w_ref[...], preferred_element_type=jnp.float32) * dinv


def _out_body(p0_ref, p1_ref, d0_ref, d1_ref, b2_ref, out_ref):
    dinv = lax.rsqrt(d0_ref[...] + d1_ref[...] + 1.0)
    o = (p0_ref[...] + p1_ref[...]) * dinv + b2_ref[...]
    col = lax.broadcasted_iota(jnp.int32, o.shape, 1)
    valid = col < C
    om = jnp.where(valid, o, -3e38)
    m = jnp.max(om, axis=1, keepdims=True)
    ex = jnp.where(valid, jnp.exp(o - m), 0.0)
    lse = jnp.log(jnp.sum(ex, axis=1, keepdims=True))
    out_ref[...] = o - m - lse


def _row_spec(width):
    return pl.BlockSpec((_BR, width), lambda i: (i, 0))


def _full_spec(shape):
    return pl.BlockSpec(shape, lambda i: tuple(0 for _ in shape))


def kernel(x, edge_index, W1, b1, W2, b2):
    f32 = jnp.float32
    x_pad = jnp.zeros((NPAD, D), f32).at[:N].set(x)
    W2p = jnp.zeros((H, 128), f32).at[:, :C].set(W2)
    b1r = b1.reshape(1, H)
    b2p = jnp.zeros((1, 128), f32).at[0, :C].set(b2)

    pad = N + jnp.arange(EP - E, dtype=jnp.int32) % (NPAD - N)
    src_flat = jnp.concatenate([edge_index[0], pad])
    dst_flat = jnp.concatenate([edge_index[1], pad])
    src1 = src_flat.reshape(SUB, CH1, B)
    dst1 = dst_flat.reshape(SUB, CH1, B)
    src2 = src_flat.reshape(2 * SUB, CH2, B)
    dst2 = dst_flat.reshape(2 * SUB, CH2, B)

    ones128 = jnp.ones((B, 128), f32)
    zer128 = jnp.zeros((B, 128), f32)

    deg0, deg1 = _deg_kernel(dst2, ones128, zer128)

    h0, h1 = pl.pallas_call(
        _mm1_body,
        grid=(NPAD // _BR,),
        in_specs=[_row_spec(D), _full_spec((D, H)), _row_spec(128),
                  _row_spec(128)],
        out_specs=[_row_spec(128), _row_spec(128)],
        out_shape=[jax.ShapeDtypeStruct((NPAD, 128), f32),
                   jax.ShapeDtypeStruct((NPAD, 128), f32)],
    )(x_pad, W1, deg0, deg1)

    a0, a1 = _agg1_kernel(h0, h1, src1, dst1)

    h2 = pl.pallas_call(
        _mm2_body,
        grid=(NPAD // _BR,),
        in_specs=[_row_spec(128), _row_spec(128), _row_spec(128),
                  _row_spec(128), _full_spec((H, 128)), _full_spec((1, H))],
        out_specs=_row_spec(128),
        out_shape=jax.ShapeDtypeStruct((NPAD, 128), f32),
    )(a0, a1, deg0, deg1, W2p, b1r)

    p0, p1 = _agg2_kernel(h2, src2, dst2, zer128)

    lsm = pl.pallas_call(
        _out_body,
        grid=(NPAD // _BR,),
        in_specs=[_row_spec(128), _row_spec(128), _row_spec(128),
                  _row_spec(128), _full_spec((1, 128))],
        out_specs=_row_spec(128),
        out_shape=jax.ShapeDtypeStruct((NPAD, 128), f32),
    )(p0, p1, deg0, deg1, b2p)

    return lsm[:N, :C]
